# baseline (speedup 1.0000x reference)
import functools

import jax
import jax.numpy as jnp
import numpy as np
from jax import lax
from jax.experimental import pallas as pl
from jax.experimental.pallas import tpu as pltpu
from jax.experimental.pallas import tpu_sc as plsc

BATCH = 16384
DIM = 32
NUM_CORES = 2
NUM_SUBCORES = 16
NUM_WORKERS = NUM_CORES * NUM_SUBCORES
B_PER_W = BATCH // NUM_WORKERS
CHUNK = 128
N_CHUNKS = B_PER_W // CHUNK

_EPS_CACHE = {}


def _eps_const(seed_int: int, shape):
    cache_key = (seed_int, shape)
    if cache_key not in _EPS_CACHE:
        try:
            cpu = jax.local_devices(backend="cpu")[0]
            with jax.default_device(cpu):
                val = np.asarray(
                    jax.random.normal(jax.random.key(seed_int), shape, jnp.float32)
                )
        except Exception:
            val = jax.random.normal(jax.random.key(seed_int), shape, jnp.float32)
        _EPS_CACHE[cache_key] = val
    return _EPS_CACHE[cache_key]


def _sc_gather(user_table, item_table, user_ids, item_ids):
    mesh = plsc.VectorSubcoreMesh(
        core_axis_name="c", subcore_axis_name="s",
        num_cores=NUM_CORES, num_subcores=NUM_SUBCORES,
    )

    @functools.partial(
        pl.kernel,
        mesh=mesh,
        compiler_params=pltpu.CompilerParams(use_tc_tiling_on_sc=False),
        out_type=[
            jax.ShapeDtypeStruct((BATCH, DIM), jnp.float32),
            jax.ShapeDtypeStruct((BATCH, DIM), jnp.float32),
        ],
        scratch_types=[
            pltpu.VMEM((B_PER_W,), jnp.int32),
            pltpu.VMEM((B_PER_W,), jnp.int32),
            pltpu.VMEM((B_PER_W, DIM), jnp.float32),
            pltpu.VMEM((B_PER_W, DIM), jnp.float32),
            pltpu.SemaphoreType.DMA,
            pltpu.SemaphoreType.DMA,
        ],
    )
    def k(ut_hbm, it_hbm, uid_hbm, iid_hbm, uout_hbm, iout_hbm,
          uidx_v, iidx_v, urows_v, irows_v, usem, isem):
        wid = lax.axis_index("s") * NUM_CORES + lax.axis_index("c")
        base = wid * B_PER_W
        pltpu.sync_copy(uid_hbm.at[pl.ds(base, B_PER_W)], uidx_v)
        pltpu.sync_copy(iid_hbm.at[pl.ds(base, B_PER_W)], iidx_v)
        copies = []
        for j in range(N_CHUNKS):
            sl = pl.ds(j * CHUNK, CHUNK)
            copies.append(
                pltpu.async_copy(ut_hbm.at[uidx_v.at[sl]], urows_v.at[sl], usem))
            copies.append(
                pltpu.async_copy(it_hbm.at[iidx_v.at[sl]], irows_v.at[sl], isem))
        for c in copies:
            c.wait()
        pltpu.sync_copy(urows_v, uout_hbm.at[pl.ds(base, B_PER_W)])
        pltpu.sync_copy(irows_v, iout_hbm.at[pl.ds(base, B_PER_W)])

    return k(user_table, item_table, user_ids, item_ids)


def _tc_dense_body(u_ref, i_ref, wum_ref, wulv_ref, wim_ref, wilv_ref,
                   bum_ref, bulv_ref, bim_ref, bilv_ref, eu_ref, ei_ref,
                   o_ref):
    u = u_ref[...]
    it = i_ref[...]
    um = jnp.dot(u, wum_ref[...], preferred_element_type=jnp.float32) + bum_ref[...]
    ulv = jnp.dot(u, wulv_ref[...], preferred_element_type=jnp.float32) + bulv_ref[...]
    im = jnp.dot(it, wim_ref[...], preferred_element_type=jnp.float32) + bim_ref[...]
    ilv = jnp.dot(it, wilv_ref[...], preferred_element_type=jnp.float32) + bilv_ref[...]
    zu = um + jnp.exp(0.5 * ulv) * eu_ref[...]
    zi = im + jnp.exp(0.5 * ilv) * ei_ref[...]
    o_ref[...] = jnp.sum(zu * zi, axis=1)


def _tc_dense(u_emb, i_emb, wum_t, wulv_t, wim_t, wilv_t,
              bum, bulv, bim, bilv, eps_u, eps_i, blk=2048):
    grid = (BATCH // blk,)
    emb_spec = pl.BlockSpec((blk, DIM), lambda b: (b, 0))
    w_spec = pl.BlockSpec((DIM, DIM), lambda b: (0, 0))
    b_spec = pl.BlockSpec((1, DIM), lambda b: (0, 0))
    return pl.pallas_call(
        _tc_dense_body,
        grid=grid,
        in_specs=[emb_spec, emb_spec,
                  w_spec, w_spec, w_spec, w_spec,
                  b_spec, b_spec, b_spec, b_spec,
                  emb_spec, emb_spec],
        out_specs=pl.BlockSpec((blk,), lambda b: (b,)),
        out_shape=jax.ShapeDtypeStruct((BATCH,), jnp.float32),
    )(u_emb, i_emb, wum_t, wulv_t, wim_t, wilv_t,
      bum, bulv, bim, bilv, eps_u, eps_i)


def kernel(user_ids, item_ids, user_table, item_table,
           W_um, b_um, W_ulv, b_ulv, W_im, b_im, W_ilv, b_ilv):
    user_ids = user_ids.astype(jnp.int32)
    item_ids = item_ids.astype(jnp.int32)
    u_emb, i_emb = _sc_gather(user_table, item_table, user_ids, item_ids)
    eps_u = jnp.asarray(_eps_const(11, (BATCH, DIM)))
    eps_i = jnp.asarray(_eps_const(13, (BATCH, DIM)))
    return _tc_dense(
        u_emb, i_emb,
        W_um.T, W_ulv.T, W_im.T, W_ilv.T,
        b_um.reshape(1, DIM), b_ulv.reshape(1, DIM),
        b_im.reshape(1, DIM), b_ilv.reshape(1, DIM),
        eps_u, eps_i,
    )
